# Initial kernel scaffold; baseline (speedup 1.0000x reference)
import functools

import jax
import jax.numpy as jnp
from jax import lax
from jax.experimental import pallas as pl
from jax.experimental.pallas import tpu as pltpu
from jax.experimental.pallas import tpu_sc as plsc

DIM = 64
PAD_ID = 0


@functools.lru_cache(maxsize=None)
def _build(B):
    info = plsc.get_sparse_core_info()
    NC, NS = info.num_cores, info.num_subcores
    NW = NC * NS
    assert B % NW == 0
    b_per_w = B // NW
    C = 1024
    assert b_per_w % C == 0
    n_chunks = b_per_w // C
    mesh = plsc.VectorSubcoreMesh(core_axis_name="c", subcore_axis_name="s")

    @functools.partial(
        pl.kernel,
        out_type=jax.ShapeDtypeStruct((B, DIM), jnp.float32),
        mesh=mesh,
        scratch_types=[
            pltpu.VMEM((b_per_w,), jnp.int32),
            pltpu.VMEM((C, DIM), jnp.float32),
            pltpu.SemaphoreType.DMA,
        ],
    )
    def k(idx_hbm, table_hbm, out_hbm, idx_v, rows_v, sem):
        wid = lax.axis_index("s") * NC + lax.axis_index("c")
        base = wid * b_per_w
        pltpu.sync_copy(idx_hbm.at[pl.ds(base, b_per_w)], idx_v)
        for j in range(n_chunks):
            pltpu.async_copy(
                table_hbm.at[idx_v.at[pl.ds(j * C, C)]], rows_v, sem
            ).wait()
            pltpu.sync_copy(rows_v, out_hbm.at[pl.ds(base + j * C, C)])

    return k


def kernel(label_ids, table):
    Bt, S = label_ids.shape
    B = Bt * S
    table = table.at[PAD_ID].set(0.0)
    idx = label_ids.reshape(B).astype(jnp.int32)
    out = _build(B)(idx, table)
    return out.reshape(Bt, S, DIM)


# speedup vs baseline: 2.6201x; 2.6201x over previous
import functools

import jax
import jax.numpy as jnp
from jax import lax
from jax.experimental import pallas as pl
from jax.experimental.pallas import tpu as pltpu
from jax.experimental.pallas import tpu_sc as plsc

DIM = 64
PAD_ID = 0


@functools.lru_cache(maxsize=None)
def _build(B):
    info = plsc.get_sparse_core_info()
    NC, NS = info.num_cores, info.num_subcores
    NW = NC * NS
    assert B % NW == 0
    b_per_w = B // NW
    C = 1024
    assert b_per_w % C == 0
    n_chunks = b_per_w // C
    mesh = plsc.VectorSubcoreMesh(core_axis_name="c", subcore_axis_name="s")

    @functools.partial(
        pl.kernel,
        out_type=jax.ShapeDtypeStruct((B, DIM), jnp.float32),
        mesh=mesh,
        compiler_params=pltpu.CompilerParams(use_tc_tiling_on_sc=False),
        scratch_types=[
            pltpu.VMEM((b_per_w,), jnp.int32),
            pltpu.VMEM((C, DIM), jnp.float32),
            pltpu.SemaphoreType.DMA,
        ],
    )
    def k(idx_hbm, table_hbm, out_hbm, idx_v, rows_v, sem):
        wid = lax.axis_index("s") * NC + lax.axis_index("c")
        base = wid * b_per_w
        pltpu.sync_copy(idx_hbm.at[pl.ds(base, b_per_w)], idx_v)
        for j in range(n_chunks):
            pltpu.async_copy(
                table_hbm.at[idx_v.at[pl.ds(j * C, C)]], rows_v, sem
            ).wait()
            pltpu.sync_copy(rows_v, out_hbm.at[pl.ds(base + j * C, C)])

    return k


def kernel(label_ids, table):
    Bt, S = label_ids.shape
    B = Bt * S
    table = table.at[PAD_ID].set(0.0)
    idx = label_ids.reshape(B).astype(jnp.int32)
    out = _build(B)(idx, table)
    return out.reshape(Bt, S, DIM)


# speedup vs baseline: 2.6264x; 1.0024x over previous
import functools

import jax
import jax.numpy as jnp
from jax import lax
from jax.experimental import pallas as pl
from jax.experimental.pallas import tpu as pltpu
from jax.experimental.pallas import tpu_sc as plsc

DIM = 64
PAD_ID = 0


@functools.lru_cache(maxsize=None)
def _build(B):
    info = plsc.get_sparse_core_info()
    NC, NS = info.num_cores, info.num_subcores
    NW = NC * NS
    assert B % NW == 0
    b_per_w = B // NW
    C = 640
    assert b_per_w % C == 0 and C % 8 == 0
    n_chunks = b_per_w // C
    mesh = plsc.VectorSubcoreMesh(core_axis_name="c", subcore_axis_name="s")

    @functools.partial(
        pl.kernel,
        out_type=jax.ShapeDtypeStruct((B, DIM), jnp.float32),
        mesh=mesh,
        compiler_params=pltpu.CompilerParams(use_tc_tiling_on_sc=False),
        scratch_types=[
            pltpu.VMEM((b_per_w,), jnp.int32),
            pltpu.VMEM((C, DIM), jnp.float32),
            pltpu.VMEM((C, DIM), jnp.float32),
            pltpu.SemaphoreType.DMA,
            pltpu.SemaphoreType.DMA,
            pltpu.SemaphoreType.DMA,
            pltpu.SemaphoreType.DMA,
        ],
    )
    def k(idx_hbm, table_hbm, out_hbm, idx_v, rows0, rows1,
          gsem0, gsem1, wsem0, wsem1):
        wid = lax.axis_index("s") * NC + lax.axis_index("c")
        base = wid * b_per_w
        pltpu.sync_copy(idx_hbm.at[pl.ds(base, b_per_w)], idx_v)
        bufs = (rows0, rows1)
        gsems = (gsem0, gsem1)
        wsems = (wsem0, wsem1)
        gh = [None] * n_chunks
        wh = [None] * n_chunks
        gh[0] = pltpu.async_copy(
            table_hbm.at[idx_v.at[pl.ds(0, C)]], rows0, gsem0)
        for j in range(n_chunks):
            p = j & 1
            q = (j + 1) & 1
            if j + 1 < n_chunks:
                if j >= 1:
                    wh[j - 1].wait()
                gh[j + 1] = pltpu.async_copy(
                    table_hbm.at[idx_v.at[pl.ds((j + 1) * C, C)]],
                    bufs[q], gsems[q])
            gh[j].wait()
            wh[j] = pltpu.async_copy(
                bufs[p], out_hbm.at[pl.ds(base + j * C, C)], wsems[p])
        wh[n_chunks - 2].wait()
        wh[n_chunks - 1].wait()

    return k


def kernel(label_ids, table):
    Bt, S = label_ids.shape
    B = Bt * S
    table = table.at[PAD_ID].set(0.0)
    idx = label_ids.reshape(B).astype(jnp.int32)
    out = _build(B)(idx, table)
    return out.reshape(Bt, S, DIM)


# speedup vs baseline: 4.9920x; 1.9007x over previous
import functools

import jax
import jax.numpy as jnp
from jax import lax
from jax.experimental import pallas as pl
from jax.experimental.pallas import tpu as pltpu
from jax.experimental.pallas import tpu_sc as plsc

DIM = 64
PAD_ID = 0


@functools.lru_cache(maxsize=None)
def _build(B, V):
    info = plsc.get_sparse_core_info()
    NC, NS = info.num_cores, info.num_subcores
    NW = NC * NS
    assert B % NW == 0
    b_per_w = B // NW
    C = 640
    assert b_per_w % C == 0 and C % 8 == 0
    n_chunks = b_per_w // C
    mesh = plsc.VectorSubcoreMesh(core_axis_name="c", subcore_axis_name="s")

    @functools.partial(
        pl.kernel,
        out_type=jax.ShapeDtypeStruct((B, DIM), jnp.float32),
        mesh=mesh,
        compiler_params=pltpu.CompilerParams(use_tc_tiling_on_sc=False),
        scratch_types=[
            pltpu.VMEM((b_per_w,), jnp.int32),
            pltpu.VMEM((C, DIM), jnp.float32),
            pltpu.VMEM((C, DIM), jnp.float32),
            pltpu.VMEM_SHARED((V, DIM), jnp.float32),
            pltpu.SemaphoreType.DMA,
            pltpu.SemaphoreType.DMA,
            pltpu.SemaphoreType.DMA,
            pltpu.SemaphoreType.DMA,
        ],
    )
    def k(idx_hbm, table_hbm, out_hbm, idx_v, rows0, rows1, table_sh,
          gsem0, gsem1, wsem0, wsem1):
        sid = lax.axis_index("s")
        wid = sid * NC + lax.axis_index("c")
        base = wid * b_per_w
        idx_cp = pltpu.async_copy(
            idx_hbm.at[pl.ds(base, b_per_w)], idx_v, gsem0)
        @pl.when(sid == 0)
        def _load_table():
            pltpu.sync_copy(table_hbm, table_sh)
        plsc.subcore_barrier()
        idx_cp.wait()
        bufs = (rows0, rows1)
        gsems = (gsem0, gsem1)
        wsems = (wsem0, wsem1)
        gh = [None] * n_chunks
        wh = [None] * n_chunks
        gh[0] = pltpu.async_copy(
            table_sh.at[idx_v.at[pl.ds(0, C)]], rows0, gsem0)
        for j in range(n_chunks):
            p = j & 1
            q = (j + 1) & 1
            if j + 1 < n_chunks:
                if j >= 1:
                    wh[j - 1].wait()
                gh[j + 1] = pltpu.async_copy(
                    table_sh.at[idx_v.at[pl.ds((j + 1) * C, C)]],
                    bufs[q], gsems[q])
            gh[j].wait()
            wh[j] = pltpu.async_copy(
                bufs[p], out_hbm.at[pl.ds(base + j * C, C)], wsems[p])
        wh[n_chunks - 2].wait()
        wh[n_chunks - 1].wait()

    return k


def kernel(label_ids, table):
    Bt, S = label_ids.shape
    B = Bt * S
    table = table.at[PAD_ID].set(0.0)
    idx = label_ids.reshape(B).astype(jnp.int32)
    out = _build(B, table.shape[0])(idx, table)
    return out.reshape(Bt, S, DIM)


# speedup vs baseline: 5.0127x; 1.0041x over previous
import functools

import jax
import jax.numpy as jnp
from jax import lax
from jax.experimental import pallas as pl
from jax.experimental.pallas import tpu as pltpu
from jax.experimental.pallas import tpu_sc as plsc

DIM = 64
PAD_ID = 0


@functools.lru_cache(maxsize=None)
def _build(Bt, S, V):
    info = plsc.get_sparse_core_info()
    NC, NS = info.num_cores, info.num_subcores
    NW = NC * NS
    assert Bt % NW == 0
    bt_per_w = Bt // NW
    b_per_w = bt_per_w * S
    C = S
    n_chunks = bt_per_w
    mesh = plsc.VectorSubcoreMesh(core_axis_name="c", subcore_axis_name="s")

    @functools.partial(
        pl.kernel,
        out_type=jax.ShapeDtypeStruct((Bt, S, DIM), jnp.float32),
        mesh=mesh,
        compiler_params=pltpu.CompilerParams(use_tc_tiling_on_sc=False),
        scratch_types=[
            pltpu.VMEM((b_per_w,), jnp.int32),
            pltpu.VMEM((C, DIM), jnp.float32),
            pltpu.VMEM((C, DIM), jnp.float32),
            pltpu.VMEM_SHARED((V, DIM), jnp.float32),
            pltpu.SemaphoreType.DMA,
            pltpu.SemaphoreType.DMA,
            pltpu.SemaphoreType.DMA,
            pltpu.SemaphoreType.DMA,
        ],
    )
    def k(idx_hbm, table_hbm, out_hbm, idx_v, rows0, rows1, table_sh,
          gsem0, gsem1, wsem0, wsem1):
        sid = lax.axis_index("s")
        wid = sid * NC + lax.axis_index("c")
        base = wid * b_per_w
        bt_base = wid * bt_per_w
        idx_cp = pltpu.async_copy(
            idx_hbm.at[pl.ds(base, b_per_w)], idx_v, gsem0)
        @pl.when(sid == 0)
        def _load_table():
            pltpu.sync_copy(table_hbm, table_sh)
        plsc.subcore_barrier()
        idx_cp.wait()
        bufs = (rows0, rows1)
        gsems = (gsem0, gsem1)
        wsems = (wsem0, wsem1)
        gh = [None] * n_chunks
        wh = [None] * n_chunks
        gh[0] = pltpu.async_copy(
            table_sh.at[idx_v.at[pl.ds(0, C)]], rows0, gsem0)
        for j in range(n_chunks):
            p = j & 1
            q = (j + 1) & 1
            if j + 1 < n_chunks:
                if j >= 1:
                    wh[j - 1].wait()
                gh[j + 1] = pltpu.async_copy(
                    table_sh.at[idx_v.at[pl.ds((j + 1) * C, C)]],
                    bufs[q], gsems[q])
            gh[j].wait()
            wh[j] = pltpu.async_copy(
                bufs[p], out_hbm.at[bt_base + j], wsems[p])
        wh[n_chunks - 2].wait()
        wh[n_chunks - 1].wait()

    return k


def kernel(label_ids, table):
    Bt, S = label_ids.shape
    table = table.at[PAD_ID].set(0.0)
    idx = label_ids.reshape(Bt * S).astype(jnp.int32)
    return _build(Bt, S, table.shape[0])(idx, table)


# speedup vs baseline: 8.2004x; 1.6359x over previous
import functools

import jax
import jax.numpy as jnp
from jax import lax
from jax.experimental import pallas as pl
from jax.experimental.pallas import tpu as pltpu
from jax.experimental.pallas import tpu_sc as plsc

DIM = 64
PAD_ID = 0


@functools.lru_cache(maxsize=None)
def _build(Bt, S, V):
    info = plsc.get_sparse_core_info()
    NC, NS = info.num_cores, info.num_subcores
    NW = NC * NS
    assert Bt % NW == 0
    bt_per_w = Bt // NW
    b_per_w = bt_per_w * S
    C = S
    n_chunks = bt_per_w
    mesh = plsc.VectorSubcoreMesh(core_axis_name="c", subcore_axis_name="s")

    @functools.partial(
        pl.kernel,
        out_type=jax.ShapeDtypeStruct((Bt, S, 128), jnp.float32),
        mesh=mesh,
        compiler_params=pltpu.CompilerParams(use_tc_tiling_on_sc=True),
        scratch_types=[
            pltpu.VMEM((b_per_w,), jnp.int32),
            pltpu.VMEM((C, 128), jnp.float32),
            pltpu.VMEM((C, 128), jnp.float32),
            pltpu.VMEM_SHARED((V, 128), jnp.float32),
            pltpu.SemaphoreType.DMA,
            pltpu.SemaphoreType.DMA,
            pltpu.SemaphoreType.DMA,
            pltpu.SemaphoreType.DMA,
        ],
    )
    def k(idx_hbm, table_hbm, out_hbm, idx_v, rows0, rows1, table_sh,
          gsem0, gsem1, wsem0, wsem1):
        sid = lax.axis_index("s")
        wid = sid * NC + lax.axis_index("c")
        base = wid * b_per_w
        bt_base = wid * bt_per_w
        idx_cp = pltpu.async_copy(
            idx_hbm.at[pl.ds(base, b_per_w)], idx_v, gsem0)
        @pl.when(sid == 0)
        def _load_table():
            pltpu.sync_copy(table_hbm, table_sh)
        plsc.subcore_barrier()
        idx_cp.wait()
        bufs = (rows0, rows1)
        gsems = (gsem0, gsem1)
        wsems = (wsem0, wsem1)
        gh = [None] * n_chunks
        wh = [None] * n_chunks
        gh[0] = pltpu.async_copy(
            table_sh.at[idx_v.at[pl.ds(0, C)]], rows0, gsem0)
        for j in range(n_chunks):
            p = j & 1
            q = (j + 1) & 1
            if j + 1 < n_chunks:
                if j >= 1:
                    wh[j - 1].wait()
                gh[j + 1] = pltpu.async_copy(
                    table_sh.at[idx_v.at[pl.ds((j + 1) * C, C)]],
                    bufs[q], gsems[q])
            gh[j].wait()
            wh[j] = pltpu.async_copy(
                bufs[p], out_hbm.at[bt_base + j], wsems[p])
        wh[n_chunks - 2].wait()
        wh[n_chunks - 1].wait()

    return k


def kernel(label_ids, table):
    Bt, S = label_ids.shape
    table = table.at[PAD_ID].set(0.0)
    table = jnp.pad(table, ((0, 0), (0, 128 - DIM)))
    idx = label_ids.reshape(Bt * S).astype(jnp.int32)
    out2 = _build(Bt, S, table.shape[0])(idx, table)
    return out2[:, :, :DIM]


# speedup vs baseline: 8.2659x; 1.0080x over previous
import functools

import jax
import jax.numpy as jnp
from jax import lax
from jax.experimental import pallas as pl
from jax.experimental.pallas import tpu as pltpu
from jax.experimental.pallas import tpu_sc as plsc

DIM = 64
PAD_ID = 0


@functools.lru_cache(maxsize=None)
def _build(Bt, S, V):
    info = plsc.get_sparse_core_info()
    NC, NS = info.num_cores, info.num_subcores
    NW = NC * NS
    assert Bt % NW == 0
    bt_per_w = Bt // NW
    b_per_w = bt_per_w * S
    C = S
    n_chunks = bt_per_w
    mesh = plsc.VectorSubcoreMesh(core_axis_name="c", subcore_axis_name="s")

    @functools.partial(
        pl.kernel,
        out_type=jax.ShapeDtypeStruct((Bt, S, 128), jnp.float32),
        mesh=mesh,
        compiler_params=pltpu.CompilerParams(use_tc_tiling_on_sc=True),
        scratch_types=[
            pltpu.VMEM((b_per_w,), jnp.int32),
            pltpu.VMEM((C, 128), jnp.float32),
            pltpu.VMEM((C, 128), jnp.float32),
            pltpu.VMEM((C, 128), jnp.float32),
            pltpu.VMEM_SHARED((V, 128), jnp.float32),
            pltpu.SemaphoreType.DMA,
            pltpu.SemaphoreType.DMA,
            pltpu.SemaphoreType.DMA,
            pltpu.SemaphoreType.DMA,
            pltpu.SemaphoreType.DMA,
            pltpu.SemaphoreType.DMA,
        ],
    )
    def k(idx_hbm, table_hbm, out_hbm, idx_v, rows0, rows1, rows2, table_sh,
          gsem0, gsem1, gsem2, wsem0, wsem1, wsem2):
        sid = lax.axis_index("s")
        wid = sid * NC + lax.axis_index("c")
        base = wid * b_per_w
        bt_base = wid * bt_per_w
        idx_cp = pltpu.async_copy(
            idx_hbm.at[pl.ds(base, b_per_w)], idx_v, gsem0)
        @pl.when(sid == 0)
        def _load_table():
            pltpu.sync_copy(table_hbm, table_sh)
        plsc.subcore_barrier()
        idx_cp.wait()
        NB = 3
        bufs = (rows0, rows1, rows2)
        gsems = (gsem0, gsem1, gsem2)
        wsems = (wsem0, wsem1, wsem2)
        gh = [None] * n_chunks
        wh = [None] * n_chunks
        for j0 in range(NB - 1):
            gh[j0] = pltpu.async_copy(
                table_sh.at[idx_v.at[pl.ds(j0 * C, C)]], bufs[j0], gsems[j0])
        for j in range(n_chunks):
            p = j % NB
            q = (j + NB - 1) % NB
            if j + NB - 1 < n_chunks:
                if j >= 1:
                    wh[j - 1].wait()
                gh[j + NB - 1] = pltpu.async_copy(
                    table_sh.at[idx_v.at[pl.ds((j + NB - 1) * C, C)]],
                    bufs[q], gsems[q])
            gh[j].wait()
            wh[j] = pltpu.async_copy(
                bufs[p], out_hbm.at[bt_base + j], wsems[p])
        for j in range(max(0, n_chunks - NB), n_chunks):
            wh[j].wait()

    return k


def kernel(label_ids, table):
    Bt, S = label_ids.shape
    table = table.at[PAD_ID].set(0.0)
    table = jnp.pad(table, ((0, 0), (0, 128 - DIM)))
    idx = label_ids.reshape(Bt * S).astype(jnp.int32)
    out2 = _build(Bt, S, table.shape[0])(idx, table)
    return out2[:, :, :DIM]
